# Initial kernel scaffold; baseline (speedup 1.0000x reference)
import jax
import jax.numpy as jnp
from jax.experimental import pallas as pl


def kernel(x, pos, edge_index, edge_attr, batch, W1, b1, W2, b2, We_w, We_b, L1w, L1b, L2w, L2b):
    raise NotImplementedError("write your pallas kernel here")


# speedup vs baseline: 7.0341x; 7.0341x over previous
import functools

import jax
import jax.numpy as jnp
import numpy as np
from jax import lax
from jax.experimental import pallas as pl
from jax.experimental.pallas import tpu as pltpu
from jax.experimental.pallas import tpu_sc as plsc

_N = 10000
_E = 320000
_DF = 128
_DE = 16
_H = 128
_EMB = 64
_NR = 32
_G = 64

_NC = 2
_NS = 16
_NW = _NC * _NS
_EPW = _E // _NW
_CH = 400
_NCHUNK = _EPW // _CH
_NGRP = _CH // 16
_SUB = 80
_NSUB = _CH // _SUB
_RPS = 624
_TAIL0 = _RPS * _NS
_TAILN = _N - _TAIL0
_FW = 48

_CENTERS = [float(c) for c in np.linspace(0.0, 5.0, _NR)]


def _sc_edge_body(px_h, py_h, pz_h, src_h, dst_h, attr_h, zf_h, za_h,
                  outf_h, outa_h,
                  px_v, py_v, pz_v, src_v, dstl_v, dst2_v, attr_v, feat_v,
                  acc_f, acc_a):
    cid = lax.axis_index("c")
    sid = lax.axis_index("s")
    wid = sid * _NC + cid

    pltpu.sync_copy(px_h, px_v)
    pltpu.sync_copy(py_h, py_v)
    pltpu.sync_copy(pz_h, pz_v)

    def _zpad(r, c):
        feat_v[r, pl.ds(32, 16)] = jnp.zeros((16,), jnp.float32)
        return c
    lax.fori_loop(0, _CH, _zpad, 0)

    r0 = sid * _RPS
    pltpu.sync_copy(zf_h.at[pl.ds(r0, _RPS), :], acc_f.at[pl.ds(r0, _RPS), :])
    pltpu.sync_copy(za_h.at[pl.ds(r0, _RPS), :], acc_a.at[pl.ds(r0, _RPS), :])

    @pl.when(sid == _NS - 1)
    def _():
        pltpu.sync_copy(zf_h.at[pl.ds(_TAIL0, _TAILN), :],
                        acc_f.at[pl.ds(_TAIL0, _TAILN), :])
        pltpu.sync_copy(za_h.at[pl.ds(_TAIL0, _TAILN), :],
                        acc_a.at[pl.ds(_TAIL0, _TAILN), :])
    plsc.subcore_barrier()

    base = wid * _EPW

    def _chunk(ci, c):
        off = base + ci * _CH
        pltpu.sync_copy(src_h.at[pl.ds(off, _CH)], src_v)
        pltpu.sync_copy(dst_h.at[pl.ds(off, _CH)], dstl_v)
        for j in range(_NSUB):
            pltpu.sync_copy(dst_h.at[pl.ds(off + j * _SUB, _SUB)], dst2_v.at[j])
        pltpu.sync_copy(attr_h.at[pl.ds(off, _CH), :], attr_v)

        def _grp(g, cc):
            e0 = g * 16
            rows = e0 + lax.iota(jnp.int32, 16)
            si = src_v[pl.ds(e0, 16)]
            di = dstl_v[pl.ds(e0, 16)]
            dx = plsc.load_gather(px_v, [di]) - plsc.load_gather(px_v, [si])
            dy = plsc.load_gather(py_v, [di]) - plsc.load_gather(py_v, [si])
            dz = plsc.load_gather(pz_v, [di]) - plsc.load_gather(pz_v, [si])
            d2 = dx * dx + dy * dy + dz * dz
            ii = plsc.bitcast(d2, jnp.int32)
            ii = jnp.int32(0x5F3759DF) - lax.shift_right_logical(ii, 1)
            y = plsc.bitcast(ii, jnp.float32)
            y = y * (1.5 - 0.5 * d2 * y * y)
            y = y * (1.5 - 0.5 * d2 * y * y)
            y = y * (1.5 - 0.5 * d2 * y * y)
            d = d2 * y
            inv = 1.0 / (d + 1e-8)
            plsc.store_scatter(feat_v, [rows, jnp.full((16,), 32, jnp.int32)], dx * inv)
            plsc.store_scatter(feat_v, [rows, jnp.full((16,), 33, jnp.int32)], dy * inv)
            plsc.store_scatter(feat_v, [rows, jnp.full((16,), 34, jnp.int32)], dz * inv)
            plsc.store_scatter(feat_v, [rows, jnp.full((16,), 35, jnp.int32)],
                               jnp.ones((16,), jnp.float32))
            for k in range(_NR):
                t = d - _CENTERS[k]
                plsc.store_scatter(feat_v, [rows, jnp.full((16,), k, jnp.int32)],
                                   jnp.exp(t * t * -10.0))
            return cc
        lax.fori_loop(0, _NGRP, _grp, 0)

        for j in range(_NSUB):
            pltpu.sync_copy(feat_v.at[pl.ds(j * _SUB, _SUB), :],
                            acc_f.at[dst2_v.at[j]], add=True)
            pltpu.sync_copy(attr_v.at[pl.ds(j * _SUB, _SUB), :],
                            acc_a.at[dst2_v.at[j]], add=True)
        return c
    lax.fori_loop(0, _NCHUNK, _chunk, 0)

    plsc.subcore_barrier()
    pltpu.sync_copy(acc_f.at[pl.ds(r0, _RPS), :], outf_h.at[cid, pl.ds(r0, _RPS), :])
    pltpu.sync_copy(acc_a.at[pl.ds(r0, _RPS), :], outa_h.at[cid, pl.ds(r0, _RPS), :])

    @pl.when(sid == _NS - 1)
    def _():
        pltpu.sync_copy(acc_f.at[pl.ds(_TAIL0, _TAILN), :],
                        outf_h.at[cid, pl.ds(_TAIL0, _TAILN), :])
        pltpu.sync_copy(acc_a.at[pl.ds(_TAIL0, _TAILN), :],
                        outa_h.at[cid, pl.ds(_TAIL0, _TAILN), :])


_BLK = 1000
_NBLK = _N // _BLK


def _tc_node_body(x_ref, af_ref, aa_ref, b3_ref, w1_ref, b1_ref, w2_ref,
                  b2_ref, wmf_ref, wma_ref, l1w_ref, l1b_ref, l2w_ref,
                  l2b_ref, out_ref):
    i = pl.program_id(0)

    @pl.when(i == 0)
    def _():
        out_ref[...] = jnp.zeros_like(out_ref)

    pf = af_ref[0] + af_ref[1]
    pa = aa_ref[0] + aa_ref[1]
    emb = (jnp.dot(pf, wmf_ref[...], preferred_element_type=jnp.float32)
           + jnp.dot(pa, wma_ref[...], preferred_element_type=jnp.float32))
    prop = (jnp.dot(emb, w2_ref[...], preferred_element_type=jnp.float32)
            + pf[:, 35:36] * b2_ref[...])
    h = (jnp.dot(x_ref[...], w1_ref[...], preferred_element_type=jnp.float32)
         + b1_ref[...] + prop)
    h = jnp.maximum(jnp.dot(h, l1w_ref[...], preferred_element_type=jnp.float32)
                    + l1b_ref[...], 0.0)
    h = jnp.maximum(jnp.dot(h, l2w_ref[...], preferred_element_type=jnp.float32)
                    + l2b_ref[...], 0.0)
    bt = b3_ref[...][0, 0, :]
    oh = (bt[None, :] == lax.broadcasted_iota(jnp.int32, (_G, _BLK), 0)
          ).astype(jnp.float32)
    out_ref[...] += jnp.dot(oh, h, preferred_element_type=jnp.float32)


def kernel(x, pos, edge_index, edge_attr, batch, W1, b1, W2, b2, We_w, We_b,
           L1w, L1b, L2w, L2b):
    src = edge_index[0]
    dst = edge_index[1]
    px = pos[:, 0]
    py = pos[:, 1]
    pz = pos[:, 2]
    zf = jnp.zeros((_N, _FW), jnp.float32)
    za = jnp.zeros((_N, _DE), jnp.float32)

    mesh = plsc.VectorSubcoreMesh(core_axis_name="c", subcore_axis_name="s")
    sc_call = pl.kernel(
        _sc_edge_body,
        mesh=mesh,
        compiler_params=pltpu.CompilerParams(needs_layout_passes=False,
                                             use_tc_tiling_on_sc=False),
        out_type=(
            jax.ShapeDtypeStruct((_NC, _N, _FW), jnp.float32),
            jax.ShapeDtypeStruct((_NC, _N, _DE), jnp.float32),
        ),
        scratch_types=[
            pltpu.VMEM((_N,), jnp.float32),
            pltpu.VMEM((_N,), jnp.float32),
            pltpu.VMEM((_N,), jnp.float32),
            pltpu.VMEM((_CH,), jnp.int32),
            pltpu.VMEM((_CH,), jnp.int32),
            pltpu.VMEM((_NSUB, _SUB), jnp.int32),
            pltpu.VMEM((_CH, _DE), jnp.float32),
            pltpu.VMEM((_CH, _FW), jnp.float32),
            pltpu.VMEM_SHARED((_N, _FW), jnp.float32),
            pltpu.VMEM_SHARED((_N, _DE), jnp.float32),
        ],
    )
    accf, acca = sc_call(px, py, pz, src, dst, edge_attr, zf, za)

    wmf = jnp.concatenate(
        [We_w[:35], We_b[None, :], jnp.zeros((_FW - 36, _EMB), jnp.float32)],
        axis=0)
    wma = We_w[35:51]
    batch3 = batch.reshape(_NBLK, 1, _BLK)

    out = pl.pallas_call(
        _tc_node_body,
        grid=(_NBLK,),
        in_specs=[
            pl.BlockSpec((_BLK, _DF), lambda i: (i, 0)),
            pl.BlockSpec((_NC, _BLK, _FW), lambda i: (0, i, 0)),
            pl.BlockSpec((_NC, _BLK, _DE), lambda i: (0, i, 0)),
            pl.BlockSpec((1, 1, _BLK), lambda i: (i, 0, 0)),
            pl.BlockSpec((_DF, _H), lambda i: (0, 0)),
            pl.BlockSpec((1, _H), lambda i: (0, 0)),
            pl.BlockSpec((_EMB, _H), lambda i: (0, 0)),
            pl.BlockSpec((1, _H), lambda i: (0, 0)),
            pl.BlockSpec((_FW, _EMB), lambda i: (0, 0)),
            pl.BlockSpec((_DE, _EMB), lambda i: (0, 0)),
            pl.BlockSpec((_H, _H), lambda i: (0, 0)),
            pl.BlockSpec((1, _H), lambda i: (0, 0)),
            pl.BlockSpec((_H, _H), lambda i: (0, 0)),
            pl.BlockSpec((1, _H), lambda i: (0, 0)),
        ],
        out_specs=pl.BlockSpec((_G, _H), lambda i: (0, 0)),
        out_shape=jax.ShapeDtypeStruct((_G, _H), jnp.float32),
    )(x, accf, acca, batch3, W1, b1.reshape(1, _H), W2, b2.reshape(1, _H),
      wmf, wma, L1w, L1b.reshape(1, _H), L2w, L2b.reshape(1, _H))
    return out


# speedup vs baseline: 10.0763x; 1.4325x over previous
import functools

import jax
import jax.numpy as jnp
import numpy as np
from jax import lax
from jax.experimental import pallas as pl
from jax.experimental.pallas import tpu as pltpu
from jax.experimental.pallas import tpu_sc as plsc

_N = 10000
_E = 320000
_DF = 128
_DE = 16
_H = 128
_EMB = 64
_NR = 32
_G = 64

_NC = 2
_NS = 16
_NW = _NC * _NS
_EPW = _E // _NW
_CH = 80
_NCHUNK = _EPW // _CH
_NGRP = _CH // 16
_SUB = 80
_NSUB = _CH // _SUB
_RPS = 624
_TAIL0 = _RPS * _NS
_TAILN = _N - _TAIL0
_FW = 48

_CENTERS = [float(c) for c in np.linspace(0.0, 5.0, _NR)]


def _sc_edge_body(px_h, py_h, pz_h, src_h, dst_h, attr_h, zf_h, za_h,
                  outf_h, outa_h,
                  px_v, py_v, pz_v,
                  src_v0, src_v1, dstl_v0, dstl_v1,
                  dst2_v0, dst2_v1, dst2_v2,
                  attr_v0, attr_v1, attr_v2,
                  feat_v0, feat_v1,
                  sem_i0, sem_i1, sem_a0, sem_a1, sem_a2,
                  sem_s0, sem_s1,
                  acc_f, acc_a):
    cid = lax.axis_index("c")
    sid = lax.axis_index("s")
    wid = sid * _NC + cid

    src_r = [src_v0, src_v1]
    dstl_r = [dstl_v0, dstl_v1]
    dst2_r = [dst2_v0, dst2_v1, dst2_v2]
    attr_r = [attr_v0, attr_v1, attr_v2]
    feat_r = [feat_v0, feat_v1]
    sem_i = [sem_i0, sem_i1]
    sem_a = [sem_a0, sem_a1, sem_a2]
    sem_s = [sem_s0, sem_s1]

    pltpu.sync_copy(px_h, px_v)
    pltpu.sync_copy(py_h, py_v)
    pltpu.sync_copy(pz_h, pz_v)

    def _zpad(r, c):
        feat_v0[r, pl.ds(32, 16)] = jnp.zeros((16,), jnp.float32)
        feat_v1[r, pl.ds(32, 16)] = jnp.zeros((16,), jnp.float32)
        return c
    lax.fori_loop(0, _CH, _zpad, 0)

    r0 = sid * _RPS
    pltpu.sync_copy(zf_h.at[pl.ds(r0, _RPS), :], acc_f.at[pl.ds(r0, _RPS), :])
    pltpu.sync_copy(za_h.at[pl.ds(r0, _RPS), :], acc_a.at[pl.ds(r0, _RPS), :])

    @pl.when(sid == _NS - 1)
    def _():
        pltpu.sync_copy(zf_h.at[pl.ds(_TAIL0, _TAILN), :],
                        acc_f.at[pl.ds(_TAIL0, _TAILN), :])
        pltpu.sync_copy(za_h.at[pl.ds(_TAIL0, _TAILN), :],
                        acc_a.at[pl.ds(_TAIL0, _TAILN), :])
    plsc.subcore_barrier()

    base = wid * _EPW

    def _issue_inputs(ci, p2, p3):
        off = base + ci * _CH
        pltpu.async_copy(src_h.at[pl.ds(off, _CH)], src_r[p2], sem_i[p2])
        pltpu.async_copy(dst_h.at[pl.ds(off, _CH)], dstl_r[p2], sem_i[p2])
        pltpu.async_copy(attr_h.at[pl.ds(off, _CH), :], attr_r[p3], sem_a[p3])
        for j in range(_NSUB):
            pltpu.async_copy(dst_h.at[pl.ds(off + j * _SUB, _SUB)],
                             dst2_r[p3].at[j], sem_a[p3])

    def _wait_idx(ci, p2):
        off = base + ci * _CH
        pltpu.make_async_copy(src_h.at[pl.ds(off, _CH)], src_r[p2],
                              sem_i[p2]).wait()
        pltpu.make_async_copy(dst_h.at[pl.ds(off, _CH)], dstl_r[p2],
                              sem_i[p2]).wait()

    def _wait_attr(ci, p3):
        off = base + ci * _CH
        pltpu.make_async_copy(attr_h.at[pl.ds(off, _CH), :], attr_r[p3],
                              sem_a[p3]).wait()
        for j in range(_NSUB):
            pltpu.make_async_copy(dst_h.at[pl.ds(off + j * _SUB, _SUB)],
                                  dst2_r[p3].at[j], sem_a[p3]).wait()

    def _compute(p2):
        src_v = src_r[p2]
        dstl_v = dstl_r[p2]
        feat_v = feat_r[p2]

        def _grp(g, cc):
            e0 = g * 16
            rows = e0 + lax.iota(jnp.int32, 16)
            si = src_v[pl.ds(e0, 16)]
            di = dstl_v[pl.ds(e0, 16)]
            dx = plsc.load_gather(px_v, [di]) - plsc.load_gather(px_v, [si])
            dy = plsc.load_gather(py_v, [di]) - plsc.load_gather(py_v, [si])
            dz = plsc.load_gather(pz_v, [di]) - plsc.load_gather(pz_v, [si])
            d2 = dx * dx + dy * dy + dz * dz
            ii = plsc.bitcast(d2, jnp.int32)
            ii = jnp.int32(0x5F3759DF) - lax.shift_right_logical(ii, 1)
            y = plsc.bitcast(ii, jnp.float32)
            y = y * (1.5 - 0.5 * d2 * y * y)
            y = y * (1.5 - 0.5 * d2 * y * y)
            y = y * (1.5 - 0.5 * d2 * y * y)
            d = d2 * y
            inv = 1.0 / (d + 1e-8)
            plsc.store_scatter(feat_v, [rows, jnp.full((16,), 32, jnp.int32)], dx * inv)
            plsc.store_scatter(feat_v, [rows, jnp.full((16,), 33, jnp.int32)], dy * inv)
            plsc.store_scatter(feat_v, [rows, jnp.full((16,), 34, jnp.int32)], dz * inv)
            plsc.store_scatter(feat_v, [rows, jnp.full((16,), 35, jnp.int32)],
                               jnp.ones((16,), jnp.float32))
            for k in range(_NR):
                t = d - _CENTERS[k]
                plsc.store_scatter(feat_v, [rows, jnp.full((16,), k, jnp.int32)],
                                   jnp.exp(t * t * -10.0))
            return cc
        lax.fori_loop(0, _NGRP, _grp, 0)

    def _issue_scatter(p2, p3):
        for j in range(_NSUB):
            pltpu.async_copy(feat_r[p2].at[pl.ds(j * _SUB, _SUB), :],
                             acc_f.at[dst2_r[p3].at[j]], sem_s[p2], add=True)
            pltpu.async_copy(attr_r[p3].at[pl.ds(j * _SUB, _SUB), :],
                             acc_a.at[dst2_r[p3].at[j]], sem_s[p2], add=True)

    def _drain_scatter(p2, p3):
        for j in range(_NSUB):
            pltpu.make_async_copy(feat_r[p2].at[pl.ds(j * _SUB, _SUB), :],
                                  acc_f.at[dst2_r[p3].at[j]], sem_s[p2]).wait()
            pltpu.make_async_copy(attr_r[p3].at[pl.ds(j * _SUB, _SUB), :],
                                  acc_a.at[dst2_r[p3].at[j]], sem_s[p2]).wait()

    def _chunk(ci, p2, p3, drain, issue_next):
        if drain:
            _drain_scatter(p2, (p3 + 1) % 3)
        if issue_next:
            _issue_inputs(ci + 1, 1 - p2, (p3 + 1) % 3)
        _wait_idx(ci, p2)
        _compute(p2)
        _wait_attr(ci, p3)
        _issue_scatter(p2, p3)

    _issue_inputs(0, 0, 0)
    _issue_inputs(1, 1, 1)
    _chunk(0, 0, 0, drain=False, issue_next=False)
    _chunk(1, 1, 1, drain=False, issue_next=True)

    def _kbody(ki, c):
        c0 = 2 + 6 * ki
        for t in range(6):
            _chunk(c0 + t, t % 2, (2 + t) % 3, drain=True, issue_next=True)
        return c
    lax.fori_loop(0, (_NCHUNK - 5) // 6, _kbody, 0)

    _chunk(_NCHUNK - 3, 0, (_NCHUNK - 3) % 3, drain=True, issue_next=True)
    _chunk(_NCHUNK - 2, 1, (_NCHUNK - 2) % 3, drain=True, issue_next=True)
    _chunk(_NCHUNK - 1, 0, (_NCHUNK - 1) % 3, drain=True, issue_next=False)
    _drain_scatter(1, (_NCHUNK - 2) % 3)
    _drain_scatter(0, (_NCHUNK - 1) % 3)

    plsc.subcore_barrier()
    pltpu.sync_copy(acc_f.at[pl.ds(r0, _RPS), :], outf_h.at[cid, pl.ds(r0, _RPS), :])
    pltpu.sync_copy(acc_a.at[pl.ds(r0, _RPS), :], outa_h.at[cid, pl.ds(r0, _RPS), :])

    @pl.when(sid == _NS - 1)
    def _():
        pltpu.sync_copy(acc_f.at[pl.ds(_TAIL0, _TAILN), :],
                        outf_h.at[cid, pl.ds(_TAIL0, _TAILN), :])
        pltpu.sync_copy(acc_a.at[pl.ds(_TAIL0, _TAILN), :],
                        outa_h.at[cid, pl.ds(_TAIL0, _TAILN), :])


_BLK = 1000
_NBLK = _N // _BLK


def _tc_node_body(x_ref, af_ref, aa_ref, b3_ref, w1_ref, b1_ref, w2_ref,
                  b2_ref, wmf_ref, wma_ref, l1w_ref, l1b_ref, l2w_ref,
                  l2b_ref, out_ref):
    i = pl.program_id(0)

    @pl.when(i == 0)
    def _():
        out_ref[...] = jnp.zeros_like(out_ref)

    pf = af_ref[0] + af_ref[1]
    pa = aa_ref[0] + aa_ref[1]
    emb = (jnp.dot(pf, wmf_ref[...], preferred_element_type=jnp.float32)
           + jnp.dot(pa, wma_ref[...], preferred_element_type=jnp.float32))
    prop = (jnp.dot(emb, w2_ref[...], preferred_element_type=jnp.float32)
            + pf[:, 35:36] * b2_ref[...])
    h = (jnp.dot(x_ref[...], w1_ref[...], preferred_element_type=jnp.float32)
         + b1_ref[...] + prop)
    h = jnp.maximum(jnp.dot(h, l1w_ref[...], preferred_element_type=jnp.float32)
                    + l1b_ref[...], 0.0)
    h = jnp.maximum(jnp.dot(h, l2w_ref[...], preferred_element_type=jnp.float32)
                    + l2b_ref[...], 0.0)
    bt = b3_ref[...][0, 0, :]
    oh = (bt[None, :] == lax.broadcasted_iota(jnp.int32, (_G, _BLK), 0)
          ).astype(jnp.float32)
    out_ref[...] += jnp.dot(oh, h, preferred_element_type=jnp.float32)


def kernel(x, pos, edge_index, edge_attr, batch, W1, b1, W2, b2, We_w, We_b,
           L1w, L1b, L2w, L2b):
    src = edge_index[0]
    dst = edge_index[1]
    px = pos[:, 0]
    py = pos[:, 1]
    pz = pos[:, 2]
    zf = jnp.zeros((_N, _FW), jnp.float32)
    za = jnp.zeros((_N, _DE), jnp.float32)

    mesh = plsc.VectorSubcoreMesh(core_axis_name="c", subcore_axis_name="s")
    sc_call = pl.kernel(
        _sc_edge_body,
        mesh=mesh,
        compiler_params=pltpu.CompilerParams(needs_layout_passes=False,
                                             use_tc_tiling_on_sc=False),
        out_type=(
            jax.ShapeDtypeStruct((_NC, _N, _FW), jnp.float32),
            jax.ShapeDtypeStruct((_NC, _N, _DE), jnp.float32),
        ),
        scratch_types=[
            pltpu.VMEM((_N,), jnp.float32),
            pltpu.VMEM((_N,), jnp.float32),
            pltpu.VMEM((_N,), jnp.float32),
            pltpu.VMEM((_CH,), jnp.int32),
            pltpu.VMEM((_CH,), jnp.int32),
            pltpu.VMEM((_CH,), jnp.int32),
            pltpu.VMEM((_CH,), jnp.int32),
            pltpu.VMEM((_NSUB, _SUB), jnp.int32),
            pltpu.VMEM((_NSUB, _SUB), jnp.int32),
            pltpu.VMEM((_NSUB, _SUB), jnp.int32),
            pltpu.VMEM((_CH, _DE), jnp.float32),
            pltpu.VMEM((_CH, _DE), jnp.float32),
            pltpu.VMEM((_CH, _DE), jnp.float32),
            pltpu.VMEM((_CH, _FW), jnp.float32),
            pltpu.VMEM((_CH, _FW), jnp.float32),
            pltpu.SemaphoreType.DMA,
            pltpu.SemaphoreType.DMA,
            pltpu.SemaphoreType.DMA,
            pltpu.SemaphoreType.DMA,
            pltpu.SemaphoreType.DMA,
            pltpu.SemaphoreType.DMA,
            pltpu.SemaphoreType.DMA,
            pltpu.VMEM_SHARED((_N, _FW), jnp.float32),
            pltpu.VMEM_SHARED((_N, _DE), jnp.float32),
        ],
    )
    accf, acca = sc_call(px, py, pz, src, dst, edge_attr, zf, za)

    wmf = jnp.concatenate(
        [We_w[:35], We_b[None, :], jnp.zeros((_FW - 36, _EMB), jnp.float32)],
        axis=0)
    wma = We_w[35:51]
    batch3 = batch.reshape(_NBLK, 1, _BLK)

    out = pl.pallas_call(
        _tc_node_body,
        grid=(_NBLK,),
        in_specs=[
            pl.BlockSpec((_BLK, _DF), lambda i: (i, 0)),
            pl.BlockSpec((_NC, _BLK, _FW), lambda i: (0, i, 0)),
            pl.BlockSpec((_NC, _BLK, _DE), lambda i: (0, i, 0)),
            pl.BlockSpec((1, 1, _BLK), lambda i: (i, 0, 0)),
            pl.BlockSpec((_DF, _H), lambda i: (0, 0)),
            pl.BlockSpec((1, _H), lambda i: (0, 0)),
            pl.BlockSpec((_EMB, _H), lambda i: (0, 0)),
            pl.BlockSpec((1, _H), lambda i: (0, 0)),
            pl.BlockSpec((_FW, _EMB), lambda i: (0, 0)),
            pl.BlockSpec((_DE, _EMB), lambda i: (0, 0)),
            pl.BlockSpec((_H, _H), lambda i: (0, 0)),
            pl.BlockSpec((1, _H), lambda i: (0, 0)),
            pl.BlockSpec((_H, _H), lambda i: (0, 0)),
            pl.BlockSpec((1, _H), lambda i: (0, 0)),
        ],
        out_specs=pl.BlockSpec((_G, _H), lambda i: (0, 0)),
        out_shape=jax.ShapeDtypeStruct((_G, _H), jnp.float32),
    )(x, accf, acca, batch3, W1, b1.reshape(1, _H), W2, b2.reshape(1, _H),
      wmf, wma, L1w, L1b.reshape(1, _H), L2w, L2b.reshape(1, _H))
    return out
